# baseline (speedup 1.0000x reference)
import jax
import jax.numpy as jnp
from jax import lax
from jax.experimental import pallas as pl
from jax.experimental.pallas import tpu as pltpu
from jax.experimental.pallas import tpu_sc as plsc

_N = 100000
_NB = 256
_BS = 1.0 / _NB
_INV_BS = float(_NB)
_MIN_SIZE = _BS * 1.4142135
_SCALE = 1.0 / (_BS * _BS * 100.0)
_NW = 32
_CHUNK = 3128
_LAST = _N - (_NW - 1) * _CHUNK
_GROUPS = -(-_CHUNK // 16)
_LAST_GROUPS = -(-_LAST // 16)
_NBINS = _NB * _NB
_KMAX = 7


def _sc_body(xy_hbm, sz_hbm, w_hbm, out_hbm, xyv, szv, wv, acc):
    wid = lax.axis_index("s") * 2 + lax.axis_index("c")
    is_last = wid == _NW - 1
    base = wid * _CHUNK

    @pl.when(jnp.logical_not(is_last))
    def _():
        pltpu.sync_copy(xy_hbm.at[pl.ds(2 * base, 2 * _CHUNK)], xyv)
        pltpu.sync_copy(sz_hbm.at[pl.ds(2 * base, 2 * _CHUNK)], szv)
        pltpu.sync_copy(w_hbm.at[pl.ds(base, _CHUNK)], wv)

    @pl.when(is_last)
    def _():
        pltpu.sync_copy(xy_hbm.at[pl.ds(2 * base, 2 * _LAST)],
                        xyv.at[pl.ds(0, 2 * _LAST)])
        pltpu.sync_copy(sz_hbm.at[pl.ds(2 * base, 2 * _LAST)],
                        szv.at[pl.ds(0, 2 * _LAST)])
        pltpu.sync_copy(w_hbm.at[pl.ds(base, _LAST)],
                        wv.at[pl.ds(0, _LAST)])

    count = jnp.where(is_last, _LAST, _CHUNK)
    ngroups = jnp.where(is_last, _LAST_GROUPS, _GROUPS)

    zero16 = jnp.zeros((16,), jnp.float32)

    def zero_body(i, c):
        acc[pl.ds(i * 16, 16)] = zero16
        return c

    lax.fori_loop(0, _NBINS // 16, zero_body, 0)

    iota = lax.iota(jnp.int32, 16)
    two_iota = iota * 2

    def group_body(g, c):
        s = g * 16
        ii = s + iota
        valid = ii < count
        i2 = 2 * s + two_iota
        x = plsc.load_gather(xyv, [i2])
        y = plsc.load_gather(xyv, [i2 + 1])
        sx = jnp.maximum(plsc.load_gather(szv, [i2]), _MIN_SIZE)
        sy = jnp.maximum(plsc.load_gather(szv, [i2 + 1]), _MIN_SIZE)
        w = wv[pl.ds(s, 16)]
        hx = 0.5 * sx
        hy = 0.5 * sy
        x_min = x - hx
        x_max = x + hx
        y_min = y - hy
        y_max = y + hy
        dens = (w * _SCALE) / (sx * sy)
        ix0 = (x_min * _INV_BS + 1024.0).astype(jnp.int32) - 1024
        iy0 = (y_min * _INV_BS + 1024.0).astype(jnp.int32) - 1024

        rowbase = []
        px = []
        mx = []
        for dx in range(_KMAX):
            bx = ix0 + dx
            lo = bx.astype(jnp.float32) * _BS
            ox = jnp.maximum(
                jnp.minimum(x_max, lo + _BS) - jnp.maximum(x_min, lo), 0.0)
            px.append(dens * ox)
            mx.append((bx >= 0) & (bx < _NB) & valid)
            rowbase.append(bx * _NB)

        col = []
        py = []
        my = []
        for dy in range(_KMAX):
            by = iy0 + dy
            lo = by.astype(jnp.float32) * _BS
            oy = jnp.maximum(
                jnp.minimum(y_max, lo + _BS) - jnp.maximum(y_min, lo), 0.0)
            py.append(oy)
            my.append((by >= 0) & (by < _NB))
            col.append(by)

        for dx in range(_KMAX):
            for dy in range(_KMAX):
                idx = rowbase[dx] + col[dy]
                val = px[dx] * py[dy]
                m = mx[dx] & my[dy]
                plsc.addupdate_scatter(acc, [idx], val, mask=m)
        return c

    lax.fori_loop(0, ngroups, group_body, 0)

    pltpu.sync_copy(acc, out_hbm.at[wid])


@jax.jit
def _sc_maps(xy, sz, w):
    mesh = plsc.VectorSubcoreMesh(core_axis_name="c", subcore_axis_name="s")
    return pl.kernel(
        _sc_body,
        out_type=jax.ShapeDtypeStruct((_NW, _NBINS), jnp.float32),
        mesh=mesh,
        compiler_params=pltpu.CompilerParams(needs_layout_passes=False),
        scratch_types=[
            pltpu.VMEM((2 * _CHUNK,), jnp.float32),
            pltpu.VMEM((2 * _CHUNK,), jnp.float32),
            pltpu.VMEM((_CHUNK,), jnp.float32),
            pltpu.VMEM((_NBINS,), jnp.float32),
        ],
    )(xy, sz, w)


def _reduce_body(maps_ref, out_ref):
    out_ref[...] = jnp.sum(maps_ref[...], axis=0)


@jax.jit
def _reduce(maps):
    return pl.pallas_call(
        _reduce_body,
        out_shape=jax.ShapeDtypeStruct((_NB, _NB), jnp.float32),
    )(maps.reshape(_NW, _NB, _NB))


def kernel(inst_sizes, inst_pos, inst_pin_weights):
    maps = _sc_maps(inst_pos.reshape(-1), inst_sizes.reshape(-1),
                    inst_pin_weights)
    return _reduce(maps)


# speedup vs baseline: 2.4333x; 2.4333x over previous
import jax
import jax.numpy as jnp
from jax import lax
from jax.experimental import pallas as pl
from jax.experimental.pallas import tpu as pltpu
from jax.experimental.pallas import tpu_sc as plsc

_N = 100000
_NB = 256
_BS = 1.0 / _NB
_INV_BS = float(_NB)
_MIN_SIZE = _BS * 1.4142135
_SCALE = 1.0 / (_BS * _BS * 100.0)
_NW = 32
_CHUNK = 3128
_LAST = _N - (_NW - 1) * _CHUNK
_GROUPS = _CHUNK // 16
_TAIL = _CHUNK - _GROUPS * 16
_NBINS = _NB * _NB
_KMAX = 7


def _sc_body(x_hbm, y_hbm, sx_hbm, sy_hbm, w_hbm, out_hbm,
             xv, yv, sxv, syv, wv, acc):
    wid = lax.axis_index("s") * 2 + lax.axis_index("c")
    is_last = wid == _NW - 1
    base = wid * _CHUNK

    zeros16 = jnp.zeros((16,), jnp.float32)
    for buf in (xv, yv, sxv, syv, wv):
        buf[pl.ds(3120, 16)] = zeros16

    @pl.when(is_last)
    def _():
        for buf in (xv, yv, sxv, syv, wv):
            for r in range(3024, 3136, 16):
                buf[pl.ds(r, 16)] = zeros16

    @pl.when(jnp.logical_not(is_last))
    def _():
        pltpu.sync_copy(x_hbm.at[pl.ds(base, _CHUNK)], xv.at[pl.ds(0, _CHUNK)])
        pltpu.sync_copy(y_hbm.at[pl.ds(base, _CHUNK)], yv.at[pl.ds(0, _CHUNK)])
        pltpu.sync_copy(sx_hbm.at[pl.ds(base, _CHUNK)], sxv.at[pl.ds(0, _CHUNK)])
        pltpu.sync_copy(sy_hbm.at[pl.ds(base, _CHUNK)], syv.at[pl.ds(0, _CHUNK)])
        pltpu.sync_copy(w_hbm.at[pl.ds(base, _CHUNK)], wv.at[pl.ds(0, _CHUNK)])

    @pl.when(is_last)
    def _():
        pltpu.sync_copy(x_hbm.at[pl.ds(base, _LAST)], xv.at[pl.ds(0, _LAST)])
        pltpu.sync_copy(y_hbm.at[pl.ds(base, _LAST)], yv.at[pl.ds(0, _LAST)])
        pltpu.sync_copy(sx_hbm.at[pl.ds(base, _LAST)], sxv.at[pl.ds(0, _LAST)])
        pltpu.sync_copy(sy_hbm.at[pl.ds(base, _LAST)], syv.at[pl.ds(0, _LAST)])
        pltpu.sync_copy(w_hbm.at[pl.ds(base, _LAST)], wv.at[pl.ds(0, _LAST)])

    zero16 = jnp.zeros((16,), jnp.float32)

    def zero_row(i, c):
        for k in range(8):
            acc[i, pl.ds(k * 16, 16)] = zero16
        return c

    lax.fori_loop(0, 512, zero_row, 0)

    def group_body(g, c):
        s = g * 16
        x = xv[pl.ds(s, 16)]
        y = yv[pl.ds(s, 16)]
        sx = jnp.maximum(sxv[pl.ds(s, 16)], _MIN_SIZE)
        sy = jnp.maximum(syv[pl.ds(s, 16)], _MIN_SIZE)
        w = wv[pl.ds(s, 16)]
        hx = 0.5 * sx
        hy = 0.5 * sy
        x_min = x - hx
        x_max = x + hx
        y_min = y - hy
        y_max = y + hy
        dens = (w * _SCALE) / (sx * sy)
        ix0 = (x_min * _INV_BS + 1024.0).astype(jnp.int32) - 1024
        iy0 = (y_min * _INV_BS + 1024.0).astype(jnp.int32) - 1024

        row2 = []
        px = []
        mx = []
        for dx in range(_KMAX):
            bx = ix0 + dx
            lo = bx.astype(jnp.float32) * _BS
            ox = jnp.maximum(
                jnp.minimum(x_max, lo + _BS) - jnp.maximum(x_min, lo), 0.0)
            px.append(dens * ox)
            mx.append((bx >= 0) & (bx < _NB))
            row2.append(bx * 2)

        hi = []
        lo_col = []
        py = []
        my = []
        for dy in range(_KMAX):
            by = iy0 + dy
            lo = by.astype(jnp.float32) * _BS
            oy = jnp.maximum(
                jnp.minimum(y_max, lo + _BS) - jnp.maximum(y_min, lo), 0.0)
            py.append(oy)
            my.append((by >= 0) & (by < _NB))
            hi.append(by >> 7)
            lo_col.append(by & 127)

        for dx in range(_KMAX):
            for dy in range(_KMAX):
                row = row2[dx] + hi[dy]
                val = px[dx] * py[dy]
                m = mx[dx] & my[dy]
                plsc.addupdate_scatter(acc, [row, lo_col[dy]], val, mask=m)
        return c

    lax.fori_loop(0, _GROUPS + 1, group_body, 0)

    pltpu.sync_copy(acc, out_hbm.at[wid])


@jax.jit
def _sc_maps(x, y, sx, sy, w):
    mesh = plsc.VectorSubcoreMesh(core_axis_name="c", subcore_axis_name="s")
    return pl.kernel(
        _sc_body,
        out_type=jax.ShapeDtypeStruct((_NW, 2 * _NB, _NB // 2), jnp.float32),
        mesh=mesh,
        compiler_params=pltpu.CompilerParams(needs_layout_passes=False),
        scratch_types=[
            pltpu.VMEM((_GROUPS * 16 + 16,), jnp.float32),
            pltpu.VMEM((_GROUPS * 16 + 16,), jnp.float32),
            pltpu.VMEM((_GROUPS * 16 + 16,), jnp.float32),
            pltpu.VMEM((_GROUPS * 16 + 16,), jnp.float32),
            pltpu.VMEM((_GROUPS * 16 + 16,), jnp.float32),
            pltpu.VMEM((2 * _NB, _NB // 2), jnp.float32),
        ],
    )(x, y, sx, sy, w)


def _reduce_body(maps_ref, out_ref):
    out_ref[...] = jnp.sum(maps_ref[...], axis=0)


@jax.jit
def _reduce(maps):
    return pl.pallas_call(
        _reduce_body,
        out_shape=jax.ShapeDtypeStruct((2 * _NB, _NB // 2), jnp.float32),
    )(maps)


def kernel(inst_sizes, inst_pos, inst_pin_weights):
    maps = _sc_maps(inst_pos[:, 0], inst_pos[:, 1],
                    inst_sizes[:, 0], inst_sizes[:, 1], inst_pin_weights)
    return _reduce(maps).reshape(_NB, _NB)
